# baseline (speedup 1.0000x reference)
import functools

import jax
import jax.numpy as jnp
from jax import lax
from jax.experimental import pallas as pl
from jax.experimental.pallas import tpu as pltpu
from jax.experimental.pallas import tpu_sc as plsc

_NBUF = 2


@functools.cache
def _build(B, D, CH):
    W = 2 * D
    mesh = plsc.VectorSubcoreMesh(core_axis_name="c", subcore_axis_name="s")
    NW = mesh.num_cores * mesh.num_subcores
    b_per_w = B // NW
    n_chunks = b_per_w // CH
    CH2 = CH // 2
    p_per_w = b_per_w // 2
    assert n_chunks % _NBUF == 0 and n_chunks >= 2 * _NBUF

    slot_scratch = []
    for _ in range(_NBUF):
        slot_scratch += [
            pltpu.VMEM((CH,), jnp.int32),
            pltpu.VMEM((CH,), jnp.int32),
            pltpu.VMEM((CH, W), jnp.float32),
            pltpu.VMEM((CH2, W), jnp.float32),
        ]

    @functools.partial(
        pl.kernel,
        mesh=mesh,
        out_type=jax.ShapeDtypeStruct((B // 2, W), jnp.float32),
        scratch_types=slot_scratch + [
            pltpu.SemaphoreType.DMA((_NBUF,)),
            pltpu.SemaphoreType.DMA((_NBUF,)),
            pltpu.SemaphoreType.DMA((_NBUF,)),
        ],
    )
    def k(idx2_hbm, off_hbm, tab2_hbm, out_hbm, *scratch):
        bufs = [scratch[4 * b: 4 * b + 4] for b in range(_NBUF)]
        isem, gsem, wsem = scratch[4 * _NBUF:]
        c = lax.axis_index("c")
        s = lax.axis_index("s")
        wid = s * mesh.num_cores + c
        base = wid * b_per_w
        obase = wid * p_per_w

        def idx_copy(i, b):
            return pltpu.make_async_copy(
                idx2_hbm.at[pl.ds(base + i * CH, CH)], bufs[b][0], isem.at[b]
            )

        def off_copy(i, b):
            return pltpu.make_async_copy(
                off_hbm.at[pl.ds(base + i * CH, CH)], bufs[b][1], isem.at[b]
            )

        def gather_copy(b):
            return pltpu.make_async_copy(
                tab2_hbm.at[bufs[b][0]], bufs[b][2], gsem.at[b]
            )

        def out_copy(i, b):
            return pltpu.make_async_copy(
                bufs[b][3], out_hbm.at[pl.ds(obase + i * CH2, CH2)], wsem.at[b]
            )

        for b in range(_NBUF):
            idx_copy(b, b).start()
            off_copy(b, b).start()

        def body(it, carry):
            g = it * _NBUF
            for b in range(_NBUF):
                i = g + b
                idx_copy(i, b).wait()
                off_copy(i, b).wait()
                gather_copy(b).start()
            for b in range(_NBUF):
                i = g + b
                off_v, g_v, ob_v = bufs[b][1], bufs[b][2], bufs[b][3]
                gather_copy(b).wait()

                @pl.when(g > 0)
                def _():
                    out_copy(i, b).wait()

                def half_select(q, carry2):
                    ovec = off_v[pl.ds(16 * q, 16)]
                    for j in range(16):
                        o = ovec[j]
                        for kk in range(4):
                            ob_v[
                                8 * q + j // 2,
                                pl.ds((j % 2) * D + 16 * kk, 16),
                            ] = g_v[16 * q + j, pl.ds(o + 16 * kk, 16)]
                    return carry2

                lax.fori_loop(0, CH // 16, half_select, 0)
                out_copy(i, b).start()

                @pl.when(g + _NBUF < n_chunks)
                def _():
                    idx_copy(i + _NBUF, b).start()
                    off_copy(i + _NBUF, b).start()

            return carry

        lax.fori_loop(0, n_chunks // _NBUF, body, 0)
        for b in range(_NBUF):
            out_copy(n_chunks - _NBUF + b, b).wait()

    return k


def kernel(input_ids, table):
    BATCH, HIST = input_ids.shape
    V, D = table.shape
    B = BATCH * HIST
    flat = input_ids.reshape(B).astype(jnp.int32)
    idx2 = lax.shift_right_logical(flat, 1)
    off64 = lax.shift_left(jnp.bitwise_and(flat, 1), 6)
    tab2 = table.reshape(V // 2, 2 * D)
    out2 = _build(B, D, 256)(idx2, off64, tab2)
    return out2.reshape(BATCH, HIST, D)


# speedup vs baseline: 1.3030x; 1.3030x over previous
import functools

import jax
import jax.numpy as jnp
from jax import lax
from jax.experimental import pallas as pl
from jax.experimental.pallas import tpu as pltpu
from jax.experimental.pallas import tpu_sc as plsc

_NBUF = 4


@functools.cache
def _build(BATCH, HIST, D, CB):
    mesh = plsc.VectorSubcoreMesh(core_axis_name="c", subcore_axis_name="s")
    NW = mesh.num_cores * mesh.num_subcores
    bat_per_w = BATCH // NW
    n_chunks = bat_per_w // CB
    assert n_chunks % _NBUF == 0 and n_chunks >= 2 * _NBUF

    slot_scratch = []
    for _ in range(_NBUF):
        slot_scratch += [
            pltpu.VMEM((CB, HIST), jnp.int32),
            pltpu.VMEM((CB, HIST, D), jnp.float32),
        ]

    @functools.partial(
        pl.kernel,
        mesh=mesh,
        out_type=jax.ShapeDtypeStruct((BATCH, HIST, D), jnp.float32),
        scratch_types=slot_scratch + [
            pltpu.SemaphoreType.DMA((_NBUF,)),
            pltpu.SemaphoreType.DMA((_NBUF,)),
            pltpu.SemaphoreType.DMA((_NBUF,)),
        ],
        compiler_params=pltpu.CompilerParams(use_tc_tiling_on_sc=False),
    )
    def k(ids_hbm, table_hbm, out_hbm, *scratch):
        bufs = [scratch[2 * b: 2 * b + 2] for b in range(_NBUF)]
        isem, gsem, wsem = scratch[2 * _NBUF:]
        c = lax.axis_index("c")
        s = lax.axis_index("s")
        wid = s * mesh.num_cores + c
        base = wid * bat_per_w

        def idx_copy(i, b):
            return pltpu.make_async_copy(
                ids_hbm.at[pl.ds(base + i * CB, CB), :], bufs[b][0], isem.at[b]
            )

        def gather_copies(b):
            return [
                pltpu.make_async_copy(
                    table_hbm.at[bufs[b][0].at[j]], bufs[b][1].at[j],
                    gsem.at[b],
                )
                for j in range(CB)
            ]

        def out_copy(i, b):
            return pltpu.make_async_copy(
                bufs[b][1],
                out_hbm.at[pl.ds(base + i * CB, CB), :, :],
                wsem.at[b],
            )

        for b in range(_NBUF):
            idx_copy(b, b).start()

        def body(it, carry):
            g = it * _NBUF
            for b in range(_NBUF):
                i = g + b
                idx_copy(i, b).wait()

                @pl.when(g > 0)
                def _():
                    out_copy(i, b).wait()

                for gc in gather_copies(b):
                    gc.start()
            for b in range(_NBUF):
                i = g + b
                for gc in gather_copies(b):
                    gc.wait()
                out_copy(i, b).start()

                @pl.when(g + _NBUF < n_chunks)
                def _():
                    idx_copy(i + _NBUF, b).start()

            return carry

        lax.fori_loop(0, n_chunks // _NBUF, body, 0)
        for b in range(_NBUF):
            out_copy(n_chunks - _NBUF + b, b).wait()

    return k


def kernel(input_ids, table):
    BATCH, HIST = input_ids.shape
    V, D = table.shape
    ids = input_ids.astype(jnp.int32)
    return _build(BATCH, HIST, D, 8)(ids, table)


# speedup vs baseline: 1.3074x; 1.0034x over previous
import functools

import jax
import jax.numpy as jnp
from jax import lax
from jax.experimental import pallas as pl
from jax.experimental.pallas import tpu as pltpu
from jax.experimental.pallas import tpu_sc as plsc

_NBUF = 4


@functools.cache
def _build(B, D, CH):
    mesh = plsc.VectorSubcoreMesh(core_axis_name="c", subcore_axis_name="s")
    NW = mesh.num_cores * mesh.num_subcores
    b_per_w = B // NW
    n_chunks = b_per_w // CH
    assert n_chunks % _NBUF == 0 and n_chunks >= 2 * _NBUF

    @functools.partial(
        pl.kernel,
        mesh=mesh,
        out_type=jax.ShapeDtypeStruct((B, D), jnp.float32),
        scratch_types=[
            pltpu.VMEM((_NBUF, CH), jnp.int32),
            pltpu.VMEM((_NBUF, CH, D), jnp.float32),
            pltpu.SemaphoreType.DMA((_NBUF,)),
            pltpu.SemaphoreType.DMA((_NBUF,)),
            pltpu.SemaphoreType.DMA((_NBUF,)),
        ],
        compiler_params=pltpu.CompilerParams(use_tc_tiling_on_sc=False),
    )
    def k(idx_hbm, table_hbm, out_hbm, idx_v, rows_v, isem, gsem, wsem):
        c = lax.axis_index("c")
        s = lax.axis_index("s")
        wid = s * mesh.num_cores + c
        base = wid * b_per_w

        def idx_copy(i, b):
            return pltpu.make_async_copy(
                idx_hbm.at[pl.ds(base + i * CH, CH)], idx_v.at[b], isem.at[b]
            )

        def gather_copy(b):
            return pltpu.make_async_copy(
                table_hbm.at[idx_v.at[b]], rows_v.at[b], gsem.at[b]
            )

        def out_copy(i, b):
            return pltpu.make_async_copy(
                rows_v.at[b], out_hbm.at[pl.ds(base + i * CH, CH)], wsem.at[b]
            )

        for b in range(_NBUF):
            idx_copy(b, b).start()

        def body(it, carry):
            g = it * _NBUF
            for b in range(_NBUF):
                i = g + b
                idx_copy(i, b).wait()

                @pl.when(g > 0)
                def _():
                    out_copy(i, b).wait()

                gather_copy(b).start()
            for b in range(_NBUF):
                i = g + b
                gather_copy(b).wait()
                out_copy(i, b).start()

                @pl.when(g + _NBUF < n_chunks)
                def _():
                    idx_copy(i + _NBUF, b).start()

            return carry

        lax.fori_loop(0, n_chunks // _NBUF, body, 0)
        for b in range(_NBUF):
            out_copy(n_chunks - _NBUF + b, b).wait()

    return k


def kernel(input_ids, table):
    BATCH, HIST = input_ids.shape
    V, D = table.shape
    B = BATCH * HIST
    flat = input_ids.reshape(B).astype(jnp.int32)
    out = _build(B, D, 400)(flat, table)
    return out.reshape(BATCH, HIST, D)
